# initial kernel scaffold (unmeasured)
import jax
import jax.numpy as jnp
from jax import lax
from jax.experimental import pallas as pl
from jax.experimental.pallas import tpu as pltpu


def kernel(
    x,
):
    def body(*refs):
        pass

    out_shape = jax.ShapeDtypeStruct(..., jnp.float32)
    return pl.pallas_call(body, out_shape=out_shape)(...)


# baseline (device time: 49360 ns/iter reference)
import jax
import jax.numpy as jnp
from jax import lax
from jax.experimental import pallas as pl
from jax.experimental.pallas import tpu as pltpu

M, N = 2048, 1024
HALF = M // 2
QTR = M // 4
EIG = M // 8


def kernel(x):
    def body(x_ref, out_ref, xb_ref, r1x_ref, r1y_ref, r2y_ref, r2x_ref,
             send_sems, recv_sems):
        mx = lax.axis_index("x")
        my = lax.axis_index("y")
        xn = (1 - mx, my)
        yn = (mx, 1 - my)

        bar = pltpu.get_barrier_semaphore()
        pl.semaphore_signal(bar, inc=1, device_id=xn,
                            device_id_type=pl.DeviceIdType.MESH)
        pl.semaphore_signal(bar, inc=1, device_id=yn,
                            device_id_type=pl.DeviceIdType.MESH)
        pl.semaphore_wait(bar, 2)

        xb_ref[...] = x_ref[0, 0].astype(jnp.bfloat16)

        base0 = mx * QTR
        base1 = HALF + my * QTR

        def exchange(idx, src, dst, dev):
            return pltpu.make_async_remote_copy(
                src_ref=src, dst_ref=dst,
                send_sem=send_sems.at[idx], recv_sem=recv_sems.at[idx],
                device_id=dev, device_id_type=pl.DeviceIdType.MESH,
            )

        rd1x = exchange(0, xb_ref.at[pl.ds((1 - mx) * QTR, QTR), :],
                        r1x_ref, xn)
        rd1y = exchange(1, xb_ref.at[pl.ds(HALF + (1 - my) * QTR, QTR), :],
                        r1y_ref, yn)
        rd1x.start()
        rd1y.start()
        rd1x.wait()
        rd1y.wait()
        out_ref[pl.ds(base0, QTR), :] = (
            xb_ref[pl.ds(base0, QTR), :] + r1x_ref[...]
        )
        out_ref[pl.ds(base1, QTR), :] = (
            xb_ref[pl.ds(base1, QTR), :] + r1y_ref[...]
        )

        e0 = base0 + my * EIG
        e1 = base1 + mx * EIG
        rd2y = exchange(2, out_ref.at[pl.ds(base0 + (1 - my) * EIG, EIG), :],
                        r2y_ref, yn)
        rd2x = exchange(3, out_ref.at[pl.ds(base1 + (1 - mx) * EIG, EIG), :],
                        r2x_ref, xn)
        rd2y.start()
        rd2x.start()
        rd2y.wait()
        rd2x.wait()
        out_ref[pl.ds(e0, EIG), :] = out_ref[pl.ds(e0, EIG), :] + r2y_ref[...]
        out_ref[pl.ds(e1, EIG), :] = out_ref[pl.ds(e1, EIG), :] + r2x_ref[...]

        rd3y = exchange(4, out_ref.at[pl.ds(e0, EIG), :],
                        out_ref.at[pl.ds(e0, EIG), :], yn)
        rd3x = exchange(5, out_ref.at[pl.ds(e1, EIG), :],
                        out_ref.at[pl.ds(e1, EIG), :], xn)
        rd3y.start()
        rd3x.start()
        rd3y.wait()
        rd3x.wait()

        rd4x = exchange(6, out_ref.at[pl.ds(base0, QTR), :],
                        out_ref.at[pl.ds(base0, QTR), :], xn)
        rd4y = exchange(7, out_ref.at[pl.ds(base1, QTR), :],
                        out_ref.at[pl.ds(base1, QTR), :], yn)
        rd4x.start()
        rd4y.start()
        rd4x.wait()
        rd4y.wait()

    return pl.pallas_call(
        body,
        out_shape=jax.ShapeDtypeStruct((M, N), jnp.bfloat16),
        in_specs=[pl.BlockSpec(memory_space=pltpu.VMEM)],
        out_specs=pl.BlockSpec(memory_space=pltpu.VMEM),
        scratch_shapes=[
            pltpu.VMEM((M, N), jnp.bfloat16),
            pltpu.VMEM((QTR, N), jnp.bfloat16),
            pltpu.VMEM((QTR, N), jnp.bfloat16),
            pltpu.VMEM((EIG, N), jnp.bfloat16),
            pltpu.VMEM((EIG, N), jnp.bfloat16),
            pltpu.SemaphoreType.DMA((8,)),
            pltpu.SemaphoreType.DMA((8,)),
        ],
        compiler_params=pltpu.CompilerParams(collective_id=0),
    )(x)


# device time: 45535 ns/iter; 1.0840x vs baseline; 1.0840x over previous
import jax
import jax.numpy as jnp
from jax import lax
from jax.experimental import pallas as pl
from jax.experimental.pallas import tpu as pltpu

M, N = 2048, 1024
HALF = M // 2
QTR = M // 4
EIG = M // 8

BF16 = jnp.bfloat16


def kernel(x):
    def body(x_ref, out_ref, sx_ref, sy_ref, r1x_ref, r1y_ref, r2y_ref,
             r2x_ref, send_sems, recv_sems):
        mx = lax.axis_index("x")
        my = lax.axis_index("y")
        xn = (1 - mx, my)
        yn = (mx, 1 - my)

        bar = pltpu.get_barrier_semaphore()
        pl.semaphore_signal(bar, inc=1, device_id=xn,
                            device_id_type=pl.DeviceIdType.MESH)
        pl.semaphore_signal(bar, inc=1, device_id=yn,
                            device_id_type=pl.DeviceIdType.MESH)
        pl.semaphore_wait(bar, 2)

        def xb(row, nrows):
            return x_ref[0, 0, pl.ds(row, nrows), :].astype(BF16)

        def exchange(idx, src, dst, dev):
            return pltpu.make_async_remote_copy(
                src_ref=src, dst_ref=dst,
                send_sem=send_sems.at[idx], recv_sem=recv_sems.at[idx],
                device_id=dev, device_id_type=pl.DeviceIdType.MESH,
            )

        base0 = mx * QTR
        base1 = HALF + my * QTR
        e0 = base0 + my * EIG
        e1 = base1 + mx * EIG
        f0 = base0 + (1 - my) * EIG
        f1 = base1 + (1 - mx) * EIG

        offA0, offB0 = (1 - my) * EIG, my * EIG
        offA1, offB1 = (1 - mx) * EIG, mx * EIG
        sx_ref[pl.ds(offA0, EIG), :] = xb((1 - mx) * QTR + offA0, EIG)
        rd0 = exchange(0, sx_ref.at[pl.ds(offA0, EIG), :],
                       r1x_ref.at[pl.ds(offA0, EIG), :], xn)
        rd0.start()
        sy_ref[pl.ds(offA1, EIG), :] = xb(HALF + (1 - my) * QTR + offA1, EIG)
        rd1 = exchange(1, sy_ref.at[pl.ds(offA1, EIG), :],
                       r1y_ref.at[pl.ds(offA1, EIG), :], yn)
        rd1.start()
        sx_ref[pl.ds(offB0, EIG), :] = xb((1 - mx) * QTR + offB0, EIG)
        rd2 = exchange(2, sx_ref.at[pl.ds(offB0, EIG), :],
                       r1x_ref.at[pl.ds(offB0, EIG), :], xn)
        rd2.start()
        sy_ref[pl.ds(offB1, EIG), :] = xb(HALF + (1 - my) * QTR + offB1, EIG)
        rd3 = exchange(3, sy_ref.at[pl.ds(offB1, EIG), :],
                       r1y_ref.at[pl.ds(offB1, EIG), :], yn)
        rd3.start()

        rd0.wait_recv()
        out_ref[pl.ds(f0, EIG), :] = xb(f0, EIG) + r1x_ref[pl.ds(offA0, EIG), :]
        rd4 = exchange(4, out_ref.at[pl.ds(f0, EIG), :], r2y_ref, yn)
        rd4.start()
        rd1.wait_recv()
        out_ref[pl.ds(f1, EIG), :] = xb(f1, EIG) + r1y_ref[pl.ds(offA1, EIG), :]
        rd5 = exchange(5, out_ref.at[pl.ds(f1, EIG), :], r2x_ref, xn)
        rd5.start()
        rd2.wait_recv()
        out_ref[pl.ds(e0, EIG), :] = xb(e0, EIG) + r1x_ref[pl.ds(offB0, EIG), :]
        rd3.wait_recv()
        out_ref[pl.ds(e1, EIG), :] = xb(e1, EIG) + r1y_ref[pl.ds(offB1, EIG), :]

        rd4.wait_recv()
        out_ref[pl.ds(e0, EIG), :] = out_ref[pl.ds(e0, EIG), :] + r2y_ref[...]
        rd6 = exchange(6, out_ref.at[pl.ds(e0, EIG), :],
                       out_ref.at[pl.ds(e0, EIG), :], yn)
        rd6.start()
        rd5.wait_recv()
        out_ref[pl.ds(e1, EIG), :] = out_ref[pl.ds(e1, EIG), :] + r2x_ref[...]
        rd7 = exchange(7, out_ref.at[pl.ds(e1, EIG), :],
                       out_ref.at[pl.ds(e1, EIG), :], xn)
        rd7.start()
        rd8 = exchange(8, out_ref.at[pl.ds(e0, EIG), :],
                       out_ref.at[pl.ds(e0, EIG), :], xn)
        rd8.start()
        rd9 = exchange(9, out_ref.at[pl.ds(e1, EIG), :],
                       out_ref.at[pl.ds(e1, EIG), :], yn)
        rd9.start()

        rd6.wait_recv()
        rd10 = exchange(10, out_ref.at[pl.ds(f0, EIG), :],
                        out_ref.at[pl.ds(f0, EIG), :], xn)
        rd10.start()
        rd7.wait_recv()
        rd11 = exchange(11, out_ref.at[pl.ds(f1, EIG), :],
                        out_ref.at[pl.ds(f1, EIG), :], yn)
        rd11.start()

        rd8.wait_recv()
        rd9.wait_recv()
        rd10.wait_recv()
        rd11.wait_recv()
        for rd in (rd0, rd1, rd2, rd3, rd4, rd5, rd6, rd7, rd8, rd9, rd10,
                   rd11):
            rd.wait_send()

    return pl.pallas_call(
        body,
        out_shape=jax.ShapeDtypeStruct((M, N), BF16),
        in_specs=[pl.BlockSpec(memory_space=pltpu.VMEM)],
        out_specs=pl.BlockSpec(memory_space=pltpu.VMEM),
        scratch_shapes=[
            pltpu.VMEM((QTR, N), BF16),
            pltpu.VMEM((QTR, N), BF16),
            pltpu.VMEM((QTR, N), BF16),
            pltpu.VMEM((QTR, N), BF16),
            pltpu.VMEM((EIG, N), BF16),
            pltpu.VMEM((EIG, N), BF16),
            pltpu.SemaphoreType.DMA((12,)),
            pltpu.SemaphoreType.DMA((12,)),
        ],
        compiler_params=pltpu.CompilerParams(collective_id=0),
    )(x)
